# baseline (speedup 1.0000x reference)
import functools

import jax
import jax.numpy as jnp
from jax import lax
from jax.experimental import pallas as pl
from jax.experimental.pallas import tpu as pltpu
from jax.experimental.pallas import tpu_sc as plsc

_NUM_EMBEDDINGS = 1000
_EMBED_DIM = 128
_BATCH = 16384

_info = plsc.get_sparse_core_info()
_NC, _NS = _info.num_cores, _info.num_subcores
_NW = _NC * _NS
_B_PER_W = _BATCH // _NW


def _make_lookup():
  mesh = plsc.VectorSubcoreMesh(core_axis_name="c", subcore_axis_name="s")

  @functools.partial(
      pl.kernel,
      mesh=mesh,
      out_type=jax.ShapeDtypeStruct((_BATCH, _EMBED_DIM), jnp.float32),
      scratch_types=[
          pltpu.VMEM((_B_PER_W,), jnp.int32),
          pltpu.VMEM((_B_PER_W, _EMBED_DIM), jnp.float32),
          pltpu.SemaphoreType.DMA,
      ],
  )
  def _lookup(table_hbm, idx_hbm, out_hbm, idx_v, rows_v, sem):
    wid = lax.axis_index("s") * _NC + lax.axis_index("c")
    base = wid * _B_PER_W
    pltpu.sync_copy(idx_hbm.at[pl.ds(base, _B_PER_W)], idx_v)
    pltpu.async_copy(table_hbm.at[idx_v], rows_v, sem).wait()
    pltpu.sync_copy(rows_v, out_hbm.at[pl.ds(base, _B_PER_W)])

  return _lookup


_lookup_call = _make_lookup()


def _tc_busy(table_ref, out_ref):
  acc = jnp.zeros((_EMBED_DIM, _EMBED_DIM), jnp.float32)
  x = table_ref[...]
  for _ in range(24):
    acc = acc + jnp.dot(x.T, x, preferred_element_type=jnp.float32)
  out_ref[...] = acc


@jax.jit
def kernel(genre_idx, genre_emb):
  idx = genre_idx.astype(jnp.int32)
  sc_out = _lookup_call(genre_emb, idx)
  tc_out = pl.pallas_call(
      _tc_busy,
      out_shape=jax.ShapeDtypeStruct((_EMBED_DIM, _EMBED_DIM), jnp.float32),
  )(genre_emb)
  return sc_out.at[0, 0].add(0.0 * tc_out[0, 0])


# speedup vs baseline: 1.2129x; 1.2129x over previous
import functools

import jax
import jax.numpy as jnp
from jax import lax
from jax.experimental import pallas as pl
from jax.experimental.pallas import tpu as pltpu
from jax.experimental.pallas import tpu_sc as plsc

_NUM_EMBEDDINGS = 1000
_EMBED_DIM = 128
_BATCH = 16384

_info = plsc.get_sparse_core_info()
_NC, _NS = _info.num_cores, _info.num_subcores
_NW = _NC * _NS
_B_PER_W = _BATCH // _NW


def _make_lookup():
  mesh = plsc.VectorSubcoreMesh(core_axis_name="c", subcore_axis_name="s")

  @functools.partial(
      pl.kernel,
      mesh=mesh,
      out_type=jax.ShapeDtypeStruct((_BATCH, _EMBED_DIM), jnp.float32),
      scratch_types=[
          pltpu.VMEM((_B_PER_W,), jnp.int32),
          pltpu.VMEM((_B_PER_W, _EMBED_DIM), jnp.float32),
          pltpu.VMEM_SHARED((_NUM_EMBEDDINGS, _EMBED_DIM), jnp.float32),
          pltpu.SemaphoreType.DMA,
      ],
  )
  def _lookup(table_hbm, idx_hbm, out_hbm, idx_v, rows_v, table_sh, sem):
    cid = lax.axis_index("c")
    sid = lax.axis_index("s")
    wid = sid * _NC + cid
    base = wid * _B_PER_W

    @pl.when(sid == 0)
    def _stage():
      pltpu.sync_copy(table_hbm, table_sh)

    pltpu.sync_copy(idx_hbm.at[pl.ds(base, _B_PER_W)], idx_v)
    plsc.subcore_barrier()
    pltpu.async_copy(table_sh.at[idx_v], rows_v, sem).wait()
    pltpu.sync_copy(rows_v, out_hbm.at[pl.ds(base, _B_PER_W)])

  return _lookup


_lookup_call = _make_lookup()


@jax.jit
def kernel(genre_idx, genre_emb):
  idx = genre_idx.astype(jnp.int32)
  return _lookup_call(genre_emb, idx)


# speedup vs baseline: 1.2461x; 1.0274x over previous
import functools

import jax
import jax.numpy as jnp
from jax import lax
from jax.experimental import pallas as pl
from jax.experimental.pallas import tpu as pltpu
from jax.experimental.pallas import tpu_sc as plsc

_NUM_EMBEDDINGS = 1000
_EMBED_DIM = 128
_BATCH = 16384

_info = plsc.get_sparse_core_info()
_NC, _NS = _info.num_cores, _info.num_subcores
_NW = _NC * _NS
_B_PER_W = _BATCH // _NW
_CHUNK = 128
_NCHUNKS = _B_PER_W // _CHUNK


def _make_lookup():
  mesh = plsc.VectorSubcoreMesh(core_axis_name="c", subcore_axis_name="s")

  scratch = [
      pltpu.VMEM((_B_PER_W,), jnp.int32),
      pltpu.VMEM_SHARED((_NUM_EMBEDDINGS, _EMBED_DIM), jnp.float32),
  ]
  scratch += [pltpu.VMEM((_CHUNK, _EMBED_DIM), jnp.float32)
              for _ in range(_NCHUNKS)]
  scratch += [pltpu.SemaphoreType.DMA for _ in range(2 * _NCHUNKS)]

  @functools.partial(
      pl.kernel,
      mesh=mesh,
      out_type=jax.ShapeDtypeStruct((_BATCH, _EMBED_DIM), jnp.float32),
      scratch_types=scratch,
  )
  def _lookup(table_hbm, idx_hbm, out_hbm, idx_v, table_sh, *bufs_and_sems):
    bufs = bufs_and_sems[:_NCHUNKS]
    gsems = bufs_and_sems[_NCHUNKS:2 * _NCHUNKS]
    ssems = bufs_and_sems[2 * _NCHUNKS:]
    cid = lax.axis_index("c")
    sid = lax.axis_index("s")
    wid = sid * _NC + cid
    base = wid * _B_PER_W

    @pl.when(sid == 0)
    def _stage():
      pltpu.sync_copy(table_hbm, table_sh)

    pltpu.sync_copy(idx_hbm.at[pl.ds(base, _B_PER_W)], idx_v)
    plsc.subcore_barrier()
    gds = [
        pltpu.async_copy(
            table_sh.at[idx_v.at[pl.ds(i * _CHUNK, _CHUNK)]],
            bufs[i], gsems[i],
        )
        for i in range(_NCHUNKS)
    ]
    sds = []
    for i in range(_NCHUNKS):
      gds[i].wait()
      sds.append(
          pltpu.async_copy(
              bufs[i], out_hbm.at[pl.ds(base + i * _CHUNK, _CHUNK)], ssems[i]
          )
      )
    for d in sds:
      d.wait()

  return _lookup


_lookup_call = _make_lookup()


@jax.jit
def kernel(genre_idx, genre_emb):
  idx = genre_idx.astype(jnp.int32)
  return _lookup_call(genre_emb, idx)


# speedup vs baseline: 1.2519x; 1.0046x over previous
import functools

import jax
import jax.numpy as jnp
from jax import lax
from jax.experimental import pallas as pl
from jax.experimental.pallas import tpu as pltpu
from jax.experimental.pallas import tpu_sc as plsc

_NUM_EMBEDDINGS = 1000
_EMBED_DIM = 128
_BATCH = 16384

_info = plsc.get_sparse_core_info()
_NC, _NS = _info.num_cores, _info.num_subcores
_NW = _NC * _NS
_B_PER_W = _BATCH // _NW
_CHUNK = 64
_NCHUNKS = _B_PER_W // _CHUNK


def _make_lookup():
  mesh = plsc.VectorSubcoreMesh(core_axis_name="c", subcore_axis_name="s")

  scratch = [
      pltpu.VMEM((_B_PER_W,), jnp.int32),
      pltpu.VMEM_SHARED((_NUM_EMBEDDINGS, _EMBED_DIM), jnp.float32),
  ]
  scratch += [pltpu.VMEM((_CHUNK, _EMBED_DIM), jnp.float32)
              for _ in range(_NCHUNKS)]
  scratch += [pltpu.SemaphoreType.DMA for _ in range(2 * _NCHUNKS)]

  @functools.partial(
      pl.kernel,
      mesh=mesh,
      out_type=jax.ShapeDtypeStruct((_BATCH, _EMBED_DIM), jnp.float32),
      scratch_types=scratch,
  )
  def _lookup(table_hbm, idx_hbm, out_hbm, idx_v, table_sh, *bufs_and_sems):
    bufs = bufs_and_sems[:_NCHUNKS]
    gsems = bufs_and_sems[_NCHUNKS:2 * _NCHUNKS]
    ssems = bufs_and_sems[2 * _NCHUNKS:]
    cid = lax.axis_index("c")
    sid = lax.axis_index("s")
    wid = sid * _NC + cid
    base = wid * _B_PER_W

    @pl.when(sid == 0)
    def _stage():
      pltpu.sync_copy(table_hbm, table_sh)

    pltpu.sync_copy(idx_hbm.at[pl.ds(base, _B_PER_W)], idx_v)
    plsc.subcore_barrier()
    gds = [
        pltpu.async_copy(
            table_sh.at[idx_v.at[pl.ds(i * _CHUNK, _CHUNK)]],
            bufs[i], gsems[i],
        )
        for i in range(_NCHUNKS)
    ]
    sds = []
    for i in range(_NCHUNKS):
      gds[i].wait()
      sds.append(
          pltpu.async_copy(
              bufs[i], out_hbm.at[pl.ds(base + i * _CHUNK, _CHUNK)], ssems[i]
          )
      )
    for d in sds:
      d.wait()

  return _lookup


_lookup_call = _make_lookup()


@jax.jit
def kernel(genre_idx, genre_emb):
  idx = genre_idx.astype(jnp.int32)
  return _lookup_call(genre_emb, idx)


# speedup vs baseline: 1.2586x; 1.0053x over previous
import functools

import jax
import jax.numpy as jnp
from jax import lax
from jax.experimental import pallas as pl
from jax.experimental.pallas import tpu as pltpu
from jax.experimental.pallas import tpu_sc as plsc

_NUM_EMBEDDINGS = 1000
_EMBED_DIM = 128
_BATCH = 16384

_info = plsc.get_sparse_core_info()
_NC, _NS = _info.num_cores, _info.num_subcores
_NW = _NC * _NS
_B_PER_W = _BATCH // _NW
_CHUNK = 64
_NCHUNKS = _B_PER_W // _CHUNK


def _make_lookup():
  mesh = plsc.VectorSubcoreMesh(core_axis_name="c", subcore_axis_name="s")

  scratch = [
      pltpu.VMEM((_B_PER_W,), jnp.int32),
      pltpu.VMEM_SHARED((_NUM_EMBEDDINGS, _EMBED_DIM), jnp.float32),
  ]
  scratch += [pltpu.VMEM((_CHUNK, _EMBED_DIM), jnp.float32)
              for _ in range(_NCHUNKS)]
  scratch += [pltpu.SemaphoreType.DMA for _ in range(2 * _NCHUNKS)]

  @functools.partial(
      pl.kernel,
      mesh=mesh,
      out_type=jax.ShapeDtypeStruct((_BATCH, _EMBED_DIM), jnp.float32),
      scratch_types=scratch,
  )
  def _lookup(table_hbm, idx_hbm, out_hbm, idx_v, table_sh, *bufs_and_sems):
    bufs = bufs_and_sems[:_NCHUNKS]
    gsems = bufs_and_sems[_NCHUNKS:2 * _NCHUNKS]
    ssems = bufs_and_sems[2 * _NCHUNKS:]
    cid = lax.axis_index("c")
    sid = lax.axis_index("s")
    wid = sid * _NC + cid
    base = wid * _B_PER_W

    @pl.when(sid < 7)
    def _stage():
      pltpu.sync_copy(
          table_hbm.at[pl.ds(sid * 128, 128)],
          table_sh.at[pl.ds(sid * 128, 128)],
      )

    @pl.when(sid == 7)
    def _stage_tail():
      pltpu.sync_copy(
          table_hbm.at[pl.ds(896, _NUM_EMBEDDINGS - 896)],
          table_sh.at[pl.ds(896, _NUM_EMBEDDINGS - 896)],
      )

    pltpu.sync_copy(idx_hbm.at[pl.ds(base, _B_PER_W)], idx_v)
    gds = [
        pltpu.async_copy(
            table_hbm.at[idx_v.at[pl.ds(0, _CHUNK)]], bufs[0], gsems[0]
        )
    ]
    plsc.subcore_barrier()
    gds += [
        pltpu.async_copy(
            table_sh.at[idx_v.at[pl.ds(i * _CHUNK, _CHUNK)]],
            bufs[i], gsems[i],
        )
        for i in range(1, _NCHUNKS)
    ]
    sds = []
    for i in range(_NCHUNKS):
      gds[i].wait()
      sds.append(
          pltpu.async_copy(
              bufs[i], out_hbm.at[pl.ds(base + i * _CHUNK, _CHUNK)], ssems[i]
          )
      )
    for d in sds:
      d.wait()

  return _lookup


_lookup_call = _make_lookup()


@jax.jit
def kernel(genre_idx, genre_emb):
  idx = genre_idx.astype(jnp.int32)
  return _lookup_call(genre_emb, idx)
